# baseline (speedup 1.0000x reference)
import functools

import jax
import jax.numpy as jnp
from jax import lax
from jax.experimental import pallas as pl
from jax.experimental.pallas import tpu as pltpu
from jax.experimental.pallas import tpu_sc as plsc

N_EDGES = 4_000_000
N_SEL = 400_000

_G = 50
_R = 625
_L = 128

_NC, _NS = 2, 16
_TH = 25_088
_TQ = _TH // 2
_SEL_PAD = _NS * _TH
_W = 1_001_472
_FS = _W // _NS
_WL = N_EDGES - 3 * _W
_FSL = 62_208
_FSLL = _WL - 15 * _FSL
_TRASH = _W
_ZB = 5_504
_WINDOWS = [
    [(0, _W), (_W, _W)],
    [(2 * _W, _W), (3 * _W, _WL)],
]


def _dense_body(m_ref, k_ref, a_ref, o_ref):
    sig = jax.nn.sigmoid(m_ref[...])
    o_ref[...] = a_ref[...] * jnp.where(k_ref[...] == 0.0, sig, 0.5)


_dense = pl.pallas_call(
    _dense_body,
    grid=(_G,),
    in_specs=[
        pl.BlockSpec((1, _R, _L), lambda i: (i, 0, 0)),
        pl.BlockSpec((1, _R, _L), lambda i: (i, 0, 0)),
        pl.BlockSpec((1, _R, _L), lambda i: (i, 0, 0)),
    ],
    out_specs=pl.BlockSpec((1, _R, _L), lambda i: (i, 0, 0)),
    out_shape=jax.ShapeDtypeStruct((_G, _R, _L), jnp.float32),
)


def _keep_body(ones_hbm, idx_hbm, keep_hbm,
               idx_v, ila_v, ilb_v, osrc_v, zbuf_v, keep_sh):
    c = lax.axis_index("c")
    s = lax.axis_index("s")
    il = [ila_v, ilb_v]

    pltpu.sync_copy(idx_hbm.at[pl.ds(s * _TH, _TH)], idx_v)
    pltpu.sync_copy(ones_hbm, osrc_v)

    def _zfill(i, carry):
        o = pl.multiple_of(i * 16, 16)
        zbuf_v[pl.ds(o, 16)] = jnp.zeros((16,), jnp.float32)
        return carry

    lax.fori_loop(0, _ZB // 16, _zfill, 0)

    for p in range(2):
        (b0, w0), (b1, w1) = _WINDOWS[p]
        base = jnp.where(c == 0, b0, b1)
        size = jnp.where(c == 0, jnp.int32(w0), jnp.int32(w1))
        last = p == 1
        fs_main = _FS if not last else None

        def _init_stream(o, ln):
            off = 0
            while off < ln:
                ck = min(_ZB, ln - off)
                pltpu.sync_copy(zbuf_v.at[pl.ds(0, ck)],
                                keep_sh.at[pl.ds(pl.multiple_of(o + off, 128), ck)])
                off += ck

        if not last:
            _init_stream(pl.multiple_of(s * _FS, 128), _FS)
        else:
            @pl.when(c == 0)
            def _init_c0():
                _init_stream(pl.multiple_of(s * _FS, 128), _FS)

            @pl.when(jnp.logical_and(c == 1, s < 15))
            def _init_c1():
                _init_stream(pl.multiple_of(s * _FSL, 128), _FSL)

            @pl.when(jnp.logical_and(c == 1, s == 15))
            def _init_c1l():
                _init_stream(15 * _FSL, _FSLL)

        for h in range(2):
            trash = _TRASH + s * 16 + lax.iota(jnp.int32, 16)

            def _remap(i, carry, base=base, size=size, h=h, trash=trash):
                o = pl.multiple_of(i * 16, 16)
                t = idx_v[pl.ds(pl.multiple_of(h * _TQ + i * 16, 16), 16)] - base
                ok = jnp.logical_and(t >= 0, t < size)
                il[h][pl.ds(o, 16)] = jnp.where(ok, t, trash)
                return carry

            lax.fori_loop(0, _TQ // 16, _remap, 0)

        plsc.subcore_barrier()
        pltpu.sync_copy(osrc_v, keep_sh.at[ila_v], add=True)
        pltpu.sync_copy(osrc_v, keep_sh.at[ilb_v], add=True)
        plsc.subcore_barrier()

        if not last:
            o_f = pl.multiple_of(s * _FS, 128)
            oh_f = pl.multiple_of(base + s * _FS, 128)
            pltpu.sync_copy(keep_sh.at[pl.ds(o_f, _FS)],
                            keep_hbm.at[pl.ds(oh_f, _FS)])
        else:
            @pl.when(c == 0)
            def _flush_c0(base=base):
                o = pl.multiple_of(s * _FS, 128)
                oh = pl.multiple_of(base + s * _FS, 128)
                pltpu.sync_copy(keep_sh.at[pl.ds(o, _FS)],
                                keep_hbm.at[pl.ds(oh, _FS)])

            @pl.when(jnp.logical_and(c == 1, s < 15))
            def _flush_c1(base=base):
                o = pl.multiple_of(s * _FSL, 128)
                oh = pl.multiple_of(base + s * _FSL, 128)
                pltpu.sync_copy(keep_sh.at[pl.ds(o, _FSL)],
                                keep_hbm.at[pl.ds(oh, _FSL)])

            @pl.when(jnp.logical_and(c == 1, s == 15))
            def _flush_c1l(base=base):
                o = 15 * _FSL
                oh = pl.multiple_of(base + 15 * _FSL, 128)
                pltpu.sync_copy(keep_sh.at[pl.ds(o, _FSLL)],
                                keep_hbm.at[pl.ds(oh, _FSLL)])


@functools.cache
def _get_keep():
    mesh = plsc.VectorSubcoreMesh(
        core_axis_name="c", subcore_axis_name="s",
        num_cores=_NC, num_subcores=_NS,
    )
    return pl.kernel(
        _keep_body,
        out_type=jax.ShapeDtypeStruct((N_EDGES,), jnp.float32),
        mesh=mesh,
        scratch_types=[
            pltpu.VMEM((_TH,), jnp.int32),
            pltpu.VMEM((_TQ,), jnp.int32),
            pltpu.VMEM((_TQ,), jnp.int32),
            pltpu.VMEM((_TQ,), jnp.float32),
            pltpu.VMEM((_ZB,), jnp.float32),
            pltpu.VMEM_SHARED((_W + 256,), jnp.float32),
        ],
    )


def kernel(mask, idx, adj_values):
    idx32 = idx.astype(jnp.int32)
    idx_pad = jnp.concatenate(
        [idx32, jnp.broadcast_to(idx32[0], (_SEL_PAD - N_SEL,))]
    )
    ones = jnp.ones((_TQ,), jnp.float32)
    keep = _get_keep()(ones, idx_pad)
    out = _dense(
        mask.reshape(_G, _R, _L),
        keep.reshape(_G, _R, _L),
        adj_values.reshape(_G, _R, _L),
    )
    return out.reshape(N_EDGES)
